# baseline (speedup 1.0000x reference)
import functools

import jax
import jax.numpy as jnp
from jax import lax
from jax.experimental import pallas as pl
from jax.experimental.pallas import tpu as pltpu
from jax.experimental.pallas import tpu_sc as plsc

_D = 64
_L = 16
_CH = 128


@functools.lru_cache(maxsize=None)
def _make_sc_kernel(N: int, T: int, V: int):
    info = plsc.get_sparse_core_info()
    nc, ns = info.num_cores, info.num_subcores
    nw = nc * ns
    npw = N // nw
    nch = npw // _CH
    assert N % nw == 0 and npw % _CH == 0 and T % npw == 0

    mesh = plsc.VectorSubcoreMesh(core_axis_name="c", subcore_axis_name="s")

    @functools.partial(
        pl.kernel,
        mesh=mesh,
        compiler_params=pltpu.CompilerParams(use_tc_tiling_on_sc=False),
        out_type=jax.ShapeDtypeStruct((N, _D), jnp.float32),
        scratch_types=[
            pltpu.VMEM((nch, _CH), jnp.int32),
            pltpu.VMEM((npw, _D), jnp.float32),
            pltpu.VMEM((npw, _D), jnp.float32),
            pltpu.SemaphoreType.DMA,
        ],
    )
    def sc_kernel(x_hbm, tok_hbm, pos_hbm, out_hbm, idx_v, rows_v, pos_v, sem):
        wid = lax.axis_index("s") * nc + lax.axis_index("c")
        base = wid * npw
        pos_base = lax.rem(base, T)
        pltpu.sync_copy(x_hbm.at[wid], idx_v)
        copies = [
            pltpu.async_copy(
                tok_hbm.at[idx_v.at[j]], rows_v.at[pl.ds(j * _CH, _CH)], sem
            )
            for j in range(nch)
        ]
        pltpu.sync_copy(pos_hbm.at[pl.ds(pos_base, npw)], pos_v)
        for cp in copies:
            cp.wait()

        def add_row(r, carry):
            for c in range(_D // _L):
                sl = pl.ds(c * _L, _L)
                rows_v[r, sl] = rows_v[r, sl] + pos_v[r, sl]
            return carry

        lax.fori_loop(0, npw, add_row, 0)
        pltpu.sync_copy(rows_v, out_hbm.at[pl.ds(base, npw)])

    return sc_kernel, nw, nch


def kernel(x, token_table, position_table):
    B, T = x.shape
    V, D = token_table.shape
    N = B * T
    sc_kernel, nw, nch = _make_sc_kernel(N, T, V)
    x_flat = x.astype(jnp.int32).reshape(nw, nch, _CH)
    out = sc_kernel(x_flat, token_table, position_table)
    return out.reshape(B, T, D)
